# baseline (speedup 1.0000x reference)
import functools

import jax
import jax.numpy as jnp
from jax import lax
from jax.experimental import pallas as pl
from jax.experimental.pallas import tpu as pltpu
from jax.experimental.pallas import tpu_sc as plsc

_NUM_EXPERTS = 8
_HIDDEN = 768
_BR = 4096

_NEG_INF = float("-inf")


def _logits_kernel(x_ref, w_ref, lg_ref):
    x = x_ref[...]
    w = w_ref[...]
    lg_ref[...] = jax.lax.dot_general(
        w, x, (((1,), (1,)), ((), ())),
        preferred_element_type=jnp.float32,
    )


def _tc_logits(xs, weight, n):
    return pl.pallas_call(
        _logits_kernel,
        grid=(n // _BR,),
        in_specs=[
            pl.BlockSpec((_BR, _HIDDEN), lambda i: (i, 0)),
            pl.BlockSpec((_NUM_EXPERTS, _HIDDEN), lambda i: (0, 0)),
        ],
        out_specs=pl.BlockSpec((_NUM_EXPERTS, _BR), lambda i: (0, i)),
        out_shape=jax.ShapeDtypeStruct((_NUM_EXPERTS, n), jnp.float32),
        compiler_params=pltpu.CompilerParams(
            dimension_semantics=("parallel",),
        ),
    )(xs, weight)


def _make_sc_gate(n):
    info = plsc.get_sparse_core_info()
    nc, ns, lanes = info.num_cores, info.num_subcores, info.num_lanes
    nw = nc * ns
    tok_w = n // nw
    groups = tok_w // lanes

    mesh = plsc.VectorSubcoreMesh(core_axis_name="c", subcore_axis_name="s")

    @functools.partial(
        pl.kernel,
        mesh=mesh,
        out_type=[
            jax.ShapeDtypeStruct((2, n), jnp.int32),
            jax.ShapeDtypeStruct((2, n), jnp.float32),
        ],
        scratch_types=[
            pltpu.VMEM((_NUM_EXPERTS, tok_w), jnp.float32),
            pltpu.VMEM((2, tok_w), jnp.int32),
            pltpu.VMEM((2, tok_w), jnp.float32),
        ],
    )
    def sc_gate(lg_hbm, idx_hbm, wgt_hbm, lg_v, idx_v, wgt_v):
        wid = lax.axis_index("s") * nc + lax.axis_index("c")
        base = wid * tok_w
        pltpu.sync_copy(lg_hbm.at[:, pl.ds(base, tok_w)], lg_v)

        def body(g, carry):
            off = g * lanes
            vs = [lg_v[e, pl.ds(off, lanes)] for e in range(_NUM_EXPERTS)]

            m = vs[0]
            for e in range(1, _NUM_EXPERTS):
                m = jnp.maximum(m, vs[e])

            i1 = jnp.zeros((lanes,), jnp.int32)
            for e in range(_NUM_EXPERTS - 1, -1, -1):
                i1 = jnp.where(vs[e] == m, jnp.full((lanes,), e, jnp.int32), i1)

            neg = jnp.full((lanes,), _NEG_INF, jnp.float32)
            ms = [
                jnp.where(i1 == jnp.full((lanes,), e, jnp.int32), neg, vs[e])
                for e in range(_NUM_EXPERTS)
            ]
            v2 = ms[0]
            for e in range(1, _NUM_EXPERTS):
                v2 = jnp.maximum(v2, ms[e])
            i2 = jnp.zeros((lanes,), jnp.int32)
            for e in range(_NUM_EXPERTS - 1, -1, -1):
                i2 = jnp.where(ms[e] == v2, jnp.full((lanes,), e, jnp.int32), i2)

            s = jnp.exp(vs[0] - m)
            for e in range(1, _NUM_EXPERTS):
                s = s + jnp.exp(vs[e] - m)

            idx_v[0, pl.ds(off, lanes)] = i1
            idx_v[1, pl.ds(off, lanes)] = i2
            wgt_v[0, pl.ds(off, lanes)] = 1.0 / s
            wgt_v[1, pl.ds(off, lanes)] = jnp.exp(v2 - m) / s
            return carry

        lax.fori_loop(0, groups, body, 0)

        pltpu.sync_copy(idx_v, idx_hbm.at[:, pl.ds(base, tok_w)])
        pltpu.sync_copy(wgt_v, wgt_hbm.at[:, pl.ds(base, tok_w)])

    return sc_gate


def kernel(x, weight):
    b, s, h = x.shape
    n = b * s
    xs = x.reshape(n, h)

    logits_t = _tc_logits(xs, weight, n)
    idx_t, wgt_t = _make_sc_gate(n)(logits_t)
    return (idx_t.T, wgt_t.T)


# speedup vs baseline: 1.5422x; 1.5422x over previous
import jax
import jax.numpy as jnp
from jax import lax
from jax.experimental import pallas as pl
from jax.experimental.pallas import tpu as pltpu

_NUM_EXPERTS = 8
_HIDDEN = 768
_BR = 2048
_DEPTH = 4


def _gate_kernel(x_hbm, w_ref, idx_ref, wgt_ref, buf, sem):
    i = pl.program_id(0)
    nb = pl.num_programs(0)

    def start(block, slot):
        pltpu.make_async_copy(
            x_hbm.at[pl.ds(block * _BR, _BR), :],
            buf.at[slot],
            sem.at[slot],
        ).start()

    @pl.when(i == 0)
    def _prologue():
        for s in range(_DEPTH):
            start(s, s)

    slot = lax.rem(i, _DEPTH)
    pltpu.make_async_copy(
        x_hbm.at[pl.ds(i * _BR, _BR), :], buf.at[slot], sem.at[slot]
    ).wait()

    x = buf[slot]
    w = w_ref[...]
    logits = jax.lax.dot_general(
        w, x, (((1,), (1,)), ((), ())),
        preferred_element_type=jnp.float32,
    )

    iota_e = jax.lax.broadcasted_iota(jnp.int32, logits.shape, 0)
    m = jnp.max(logits, axis=0, keepdims=True)
    e = jnp.exp(logits - m)
    s_den = jnp.sum(e, axis=0, keepdims=True)

    i1 = jnp.min(jnp.where(logits == m, iota_e, _NUM_EXPERTS), axis=0, keepdims=True)
    masked = jnp.where(iota_e == i1, -jnp.inf, logits)
    v2 = jnp.max(masked, axis=0, keepdims=True)
    i2 = jnp.min(jnp.where(masked == v2, iota_e, _NUM_EXPERTS), axis=0, keepdims=True)

    p1 = 1.0 / s_den
    p2 = jnp.exp(v2 - m) / s_den

    idx_ref[...] = jnp.concatenate([i1, i2], axis=0)
    wgt_ref[...] = jnp.concatenate([p1, p2], axis=0)

    nxt = i + _DEPTH

    @pl.when(nxt < nb)
    def _refill():
        pltpu.make_async_copy(
            x_hbm.at[pl.ds(nxt * _BR, _BR), :], buf.at[slot], sem.at[slot]
        ).start()


def kernel(x, weight):
    b, s, h = x.shape
    n = b * s
    xs = x.reshape(n, h)

    grid = (n // _BR,)
    idx_t, wgt_t = pl.pallas_call(
        _gate_kernel,
        grid=grid,
        in_specs=[
            pl.BlockSpec(memory_space=pl.ANY),
            pl.BlockSpec((_NUM_EXPERTS, h), lambda i: (0, 0)),
        ],
        out_specs=[
            pl.BlockSpec((2, _BR), lambda i: (0, i)),
            pl.BlockSpec((2, _BR), lambda i: (0, i)),
        ],
        out_shape=[
            jax.ShapeDtypeStruct((2, n), jnp.int32),
            jax.ShapeDtypeStruct((2, n), jnp.float32),
        ],
        scratch_shapes=[
            pltpu.VMEM((_DEPTH, _BR, _HIDDEN), jnp.float32),
            pltpu.SemaphoreType.DMA((_DEPTH,)),
        ],
    )(xs, weight)
    return (idx_t.T, wgt_t.T)
